# Initial kernel scaffold; baseline (speedup 1.0000x reference)
import jax
import jax.numpy as jnp
from jax.experimental import pallas as pl


def kernel(x, edge_attr, W, b, edge_index):
    raise NotImplementedError("write your pallas kernel here")


# speedup vs baseline: 4.8149x; 4.8149x over previous
import functools

import jax
import jax.numpy as jnp
from jax import lax
from jax.experimental import pallas as pl
from jax.experimental.pallas import tpu as pltpu
from jax.experimental.pallas import tpu_sc as plsc

_N = 10000
_E = 320000
_DN = 128
_DE = 16
_DO = 128
_CH = 80
_ER = _E // _CH
_NC, _NS = 2, 16
_RT_A = _ER // _NS
_RT_C = _ER // (_NC * _NS)
_NT = _N // _NS

_mesh = plsc.VectorSubcoreMesh(core_axis_name="c", subcore_axis_name="s")

_f32 = jnp.float32

_sc_params = pltpu.CompilerParams(use_tc_tiling_on_sc=False)


@functools.partial(
    pl.kernel,
    mesh=_mesh,
    out_type=[jax.ShapeDtypeStruct((_N, _DE), _f32)] * 3,
    scratch_types=[
        pltpu.VMEM((_RT_A, _CH), jnp.int32),
        pltpu.VMEM((_RT_A, _CH), jnp.int32),
        pltpu.VMEM((_CH, _DE), _f32),
        pltpu.VMEM((_CH, _DE), _f32),
        pltpu.VMEM((_CH, _DE), _f32),
        pltpu.VMEM((_NT, _DE), _f32),
        pltpu.VMEM_SHARED((_N, _DE), _f32),
        pltpu.VMEM_SHARED((_N, _DE), _f32),
        pltpu.VMEM_SHARED((_N, _DE), _f32),
    ],
    compiler_params=_sc_params,
)
def _seg(dst_hbm, src_hbm, ea_hbm, h1_hbm, h2_hbm, dg_hbm,
         dstv, srcv, eav, onesv, gatv, stv, h1s, h2s, dgs):
    cid = lax.axis_index("c")
    sid = lax.axis_index("s")

    @pl.when(cid == 0)
    def _():
        @pl.loop(0, _NT)
        def _(i):
            stv[i, :] = jnp.zeros((_DE,), _f32)

        @pl.loop(0, _CH)
        def _(i):
            onesv[i, :] = jnp.ones((_DE,), _f32)

        nbase = sid * _NT
        pltpu.sync_copy(stv, h1s.at[pl.ds(nbase, _NT)])
        pltpu.sync_copy(stv, h2s.at[pl.ds(nbase, _NT)])
        pltpu.sync_copy(stv, dgs.at[pl.ds(nbase, _NT)])

        rbase = sid * _RT_A
        pltpu.sync_copy(dst_hbm.at[pl.ds(rbase, _RT_A)], dstv)
        pltpu.sync_copy(src_hbm.at[pl.ds(rbase, _RT_A)], srcv)
        plsc.subcore_barrier()

        @pl.loop(0, _RT_A)
        def _(c):
            pltpu.sync_copy(ea_hbm.at[pl.ds((rbase + c) * _CH, _CH)], eav)
            pltpu.sync_copy(eav, h1s.at[dstv.at[c]], add=True)
            pltpu.sync_copy(onesv, dgs.at[dstv.at[c]], add=True)

        plsc.subcore_barrier()

        @pl.loop(0, _RT_A)
        def _(c):
            pltpu.sync_copy(h1s.at[srcv.at[c]], gatv)
            pltpu.sync_copy(gatv, h2s.at[dstv.at[c]], add=True)

        plsc.subcore_barrier()

        pltpu.sync_copy(h1s.at[pl.ds(nbase, _NT)], stv)
        pltpu.sync_copy(stv, h1_hbm.at[pl.ds(nbase, _NT)])
        pltpu.sync_copy(h2s.at[pl.ds(nbase, _NT)], stv)
        pltpu.sync_copy(stv, h2_hbm.at[pl.ds(nbase, _NT)])
        pltpu.sync_copy(dgs.at[pl.ds(nbase, _NT)], stv)
        pltpu.sync_copy(stv, dg_hbm.at[pl.ds(nbase, _NT)])


def _mm_body(x_ref, h1_ref, h2_ref, dg_ref, wx_ref, w1_ref, w2_ref, b_ref, o_ref):
    norm = lax.rsqrt(jnp.maximum(dg_ref[:, 0:1], 1.0))
    acc = jnp.dot(x_ref[...], wx_ref[...],
                  preferred_element_type=_f32, precision=lax.Precision.HIGHEST)
    acc = acc + jnp.dot(h1_ref[...] * norm, w1_ref[...],
                        preferred_element_type=_f32, precision=lax.Precision.HIGHEST)
    acc = acc + jnp.dot(h2_ref[...] * norm, w2_ref[...],
                        preferred_element_type=_f32, precision=lax.Precision.HIGHEST)
    o_ref[...] = acc + b_ref[...]


_BLK = 2000


def _mm(x, h1, h2, dg, wx, w1, w2, b2):
    return pl.pallas_call(
        _mm_body,
        grid=(_N // _BLK,),
        in_specs=[
            pl.BlockSpec((_BLK, _DN), lambda i: (i, 0)),
            pl.BlockSpec((_BLK, _DE), lambda i: (i, 0)),
            pl.BlockSpec((_BLK, _DE), lambda i: (i, 0)),
            pl.BlockSpec((_BLK, _DE), lambda i: (i, 0)),
            pl.BlockSpec((_DN, _DO), lambda i: (0, 0)),
            pl.BlockSpec((_DE, _DO), lambda i: (0, 0)),
            pl.BlockSpec((_DE, _DO), lambda i: (0, 0)),
            pl.BlockSpec((1, _DO), lambda i: (0, 0)),
        ],
        out_specs=pl.BlockSpec((_BLK, _DO), lambda i: (i, 0)),
        out_shape=jax.ShapeDtypeStruct((_N, _DO), _f32),
    )(x, h1, h2, dg, wx, w1, w2, b2)


@functools.partial(
    pl.kernel,
    mesh=_mesh,
    out_type=jax.ShapeDtypeStruct((_E, _DO), _f32),
    scratch_types=[
        pltpu.VMEM((_RT_C, _CH), jnp.int32),
        pltpu.VMEM((_CH, _DO), _f32),
    ],
    compiler_params=_sc_params,
)
def _gat(src_hbm, rst_hbm, est_hbm, srcv, bufv):
    wid = lax.axis_index("s") * _NC + lax.axis_index("c")
    rbase = wid * _RT_C
    pltpu.sync_copy(src_hbm.at[pl.ds(rbase, _RT_C)], srcv)

    @pl.loop(0, _RT_C)
    def _(c):
        pltpu.sync_copy(rst_hbm.at[srcv.at[c]], bufv)
        pltpu.sync_copy(bufv, est_hbm.at[pl.ds((rbase + c) * _CH, _CH)])


def kernel(x, edge_attr, W, b, edge_index):
    src2d = edge_index[0].reshape(_ER, _CH)
    dst2d = edge_index[1].reshape(_ER, _CH)
    h1, h2, dg = _seg(dst2d, src2d, edge_attr)
    wx = W[:, :_DN].T
    w1 = W[:, _DN:_DN + _DE].T
    w2 = W[:, _DN + _DE:].T
    rst = _mm(x, h1, h2, dg, wx, w1, w2, b.reshape(1, _DO))
    est = _gat(src2d, rst)
    return (rst, est)


# speedup vs baseline: 5.5286x; 1.1482x over previous
import functools

import jax
import jax.numpy as jnp
from jax import lax
from jax.experimental import pallas as pl
from jax.experimental.pallas import tpu as pltpu
from jax.experimental.pallas import tpu_sc as plsc

_N = 10000
_E = 320000
_DN = 128
_DE = 16
_DO = 128
_CH = 80
_ER = _E // _CH
_NC, _NS = 2, 16
_RT_A = _ER // _NS
_RT_C = _ER // (_NC * _NS)
_NT = _N // _NS

_mesh = plsc.VectorSubcoreMesh(core_axis_name="c", subcore_axis_name="s")

_f32 = jnp.float32

_sc_params = pltpu.CompilerParams(use_tc_tiling_on_sc=False)


@functools.partial(
    pl.kernel,
    mesh=_mesh,
    out_type=[jax.ShapeDtypeStruct((_N, _DE), _f32)] * 3,
    scratch_types=[
        pltpu.VMEM((_RT_A, _CH), jnp.int32),
        pltpu.VMEM((_RT_A, _CH), jnp.int32),
        pltpu.VMEM((_CH, _DE), _f32),
        pltpu.VMEM((_CH, _DE), _f32),
        pltpu.VMEM((_CH, _DE), _f32),
        pltpu.VMEM((_NT, _DE), _f32),
        pltpu.VMEM_SHARED((_N, _DE), _f32),
        pltpu.VMEM_SHARED((_N, _DE), _f32),
        pltpu.VMEM_SHARED((_N, _DE), _f32),
    ],
    compiler_params=_sc_params,
)
def _seg(ei_hbm, ea_hbm, h1_hbm, h2_hbm, dg_hbm,
         dstv, srcv, eav, onesv, gatv, stv, h1s, h2s, dgs):
    cid = lax.axis_index("c")
    sid = lax.axis_index("s")

    @pl.when(cid == 0)
    def _():
        @pl.loop(0, _NT)
        def _(i):
            stv[i, :] = jnp.zeros((_DE,), _f32)

        @pl.loop(0, _CH)
        def _(i):
            onesv[i, :] = jnp.ones((_DE,), _f32)

        nbase = sid * _NT
        pltpu.sync_copy(stv, h1s.at[pl.ds(nbase, _NT)])
        pltpu.sync_copy(stv, h2s.at[pl.ds(nbase, _NT)])
        pltpu.sync_copy(stv, dgs.at[pl.ds(nbase, _NT)])

        rbase = sid * _RT_A
        pltpu.sync_copy(ei_hbm.at[1, pl.ds(rbase, _RT_A)], dstv)
        pltpu.sync_copy(ei_hbm.at[0, pl.ds(rbase, _RT_A)], srcv)
        plsc.subcore_barrier()

        @pl.loop(0, _RT_A)
        def _(c):
            pltpu.sync_copy(ea_hbm.at[pl.ds((rbase + c) * _CH, _CH)], eav)
            pltpu.sync_copy(eav, h1s.at[dstv.at[c]], add=True)
            pltpu.sync_copy(onesv, dgs.at[dstv.at[c]], add=True)

        plsc.subcore_barrier()

        @pl.loop(0, _RT_A)
        def _(c):
            pltpu.sync_copy(h1s.at[srcv.at[c]], gatv)
            pltpu.sync_copy(gatv, h2s.at[dstv.at[c]], add=True)

        plsc.subcore_barrier()

        pltpu.sync_copy(h1s.at[pl.ds(nbase, _NT)], stv)
        pltpu.sync_copy(stv, h1_hbm.at[pl.ds(nbase, _NT)])
        pltpu.sync_copy(h2s.at[pl.ds(nbase, _NT)], stv)
        pltpu.sync_copy(stv, h2_hbm.at[pl.ds(nbase, _NT)])
        pltpu.sync_copy(dgs.at[pl.ds(nbase, _NT)], stv)
        pltpu.sync_copy(stv, dg_hbm.at[pl.ds(nbase, _NT)])


def _mm_body(x_ref, h1_ref, h2_ref, dg_ref, wx_ref, w1_ref, w2_ref, b_ref, o_ref):
    norm = lax.rsqrt(jnp.maximum(dg_ref[:, 0:1], 1.0))
    acc = jnp.dot(x_ref[...], wx_ref[...],
                  preferred_element_type=_f32, precision=lax.Precision.HIGHEST)
    acc = acc + jnp.dot(h1_ref[...] * norm, w1_ref[...],
                        preferred_element_type=_f32, precision=lax.Precision.HIGHEST)
    acc = acc + jnp.dot(h2_ref[...] * norm, w2_ref[...],
                        preferred_element_type=_f32, precision=lax.Precision.HIGHEST)
    o_ref[...] = acc + b_ref[...]


_BLK = 2000


def _mm(x, h1, h2, dg, wx, w1, w2, b2):
    return pl.pallas_call(
        _mm_body,
        grid=(_N // _BLK,),
        in_specs=[
            pl.BlockSpec((_BLK, _DN), lambda i: (i, 0)),
            pl.BlockSpec((_BLK, _DE), lambda i: (i, 0)),
            pl.BlockSpec((_BLK, _DE), lambda i: (i, 0)),
            pl.BlockSpec((_BLK, _DE), lambda i: (i, 0)),
            pl.BlockSpec((_DN, _DO), lambda i: (0, 0)),
            pl.BlockSpec((_DE, _DO), lambda i: (0, 0)),
            pl.BlockSpec((_DE, _DO), lambda i: (0, 0)),
            pl.BlockSpec((1, _DO), lambda i: (0, 0)),
        ],
        out_specs=pl.BlockSpec((_BLK, _DO), lambda i: (i, 0)),
        out_shape=jax.ShapeDtypeStruct((_N, _DO), _f32),
    )(x, h1, h2, dg, wx, w1, w2, b2)


_G = 5
_NG = _RT_C // _G
_GE = _G * _CH


@functools.partial(
    pl.kernel,
    mesh=_mesh,
    out_type=jax.ShapeDtypeStruct((_E, _DO), _f32),
    scratch_types=[
        pltpu.VMEM((_RT_C, _CH), jnp.int32),
        pltpu.VMEM((_GE, _DO), _f32),
        pltpu.VMEM((_GE, _DO), _f32),
        pltpu.SemaphoreType.DMA,
        pltpu.SemaphoreType.DMA,
        pltpu.SemaphoreType.DMA,
    ],
    compiler_params=_sc_params,
)
def _gat(ei_hbm, rst_hbm, est_hbm, srcv, buf0, buf1, gsem, wsem0, wsem1):
    wid = lax.axis_index("s") * _NC + lax.axis_index("c")
    rbase = wid * _RT_C
    pltpu.sync_copy(ei_hbm.at[0, pl.ds(rbase, _RT_C)], srcv)
    bufs = (buf0, buf1)
    wsems = (wsem0, wsem1)
    whandles = [None, None]
    for g in range(_NG):
        b = g % 2
        buf = bufs[b]
        if whandles[b] is not None:
            whandles[b].wait()
        ghandles = []
        for j in range(_G):
            ghandles.append(pltpu.async_copy(
                rst_hbm.at[srcv.at[g * _G + j]],
                buf.at[pl.ds(j * _CH, _CH)], gsem))
        for h in ghandles:
            h.wait()
        whandles[b] = pltpu.async_copy(
            buf, est_hbm.at[pl.ds((rbase + g * _G) * _CH, _GE)], wsems[b])
    whandles[0].wait()
    whandles[1].wait()


def kernel(x, edge_attr, W, b, edge_index):
    ei3 = edge_index.reshape(2, _ER, _CH)
    h1, h2, dg = _seg(ei3, edge_attr)
    wx = W[:, :_DN].T
    w1 = W[:, _DN:_DN + _DE].T
    w2 = W[:, _DN + _DE:].T
    rst = _mm(x, h1, h2, dg, wx, w1, w2, b.reshape(1, _DO))
    est = _gat(ei3, rst)
    return (rst, est)


# speedup vs baseline: 8.6825x; 1.5705x over previous
import functools

import jax
import jax.numpy as jnp
from jax import lax
from jax.experimental import pallas as pl
from jax.experimental.pallas import tpu as pltpu
from jax.experimental.pallas import tpu_sc as plsc

_N = 10000
_E = 320000
_DN = 128
_DE = 16
_DO = 128
_CH = 80
_ER = _E // _CH
_NC, _NS = 2, 16
_RT_A = _ER // _NS
_RT_C = _ER // (_NC * _NS)
_NT = _N // _NS

_mesh = plsc.VectorSubcoreMesh(core_axis_name="c", subcore_axis_name="s")

_f32 = jnp.float32

_sc_params = pltpu.CompilerParams(use_tc_tiling_on_sc=False)


_G1 = 25
_NG1 = _RT_A // _G1
_GE1 = _G1 * _CH


@functools.partial(
    pl.kernel,
    mesh=_mesh,
    out_type=[jax.ShapeDtypeStruct((_N, _DE), _f32),
              jax.ShapeDtypeStruct((_N, _DE), _f32),
              jax.ShapeDtypeStruct((_N,), _f32)],
    scratch_types=[
        pltpu.VMEM((_RT_A, _CH), jnp.int32),
        pltpu.VMEM((_RT_A, _CH), jnp.int32),
        pltpu.VMEM((_GE1, _DE), _f32),
        pltpu.VMEM((_GE1, _DE), _f32),
        pltpu.VMEM((_CH,), _f32),
        pltpu.VMEM((1024,), _f32),
        pltpu.VMEM_SHARED((_N, _DE), _f32),
        pltpu.VMEM_SHARED((_N, _DE), _f32),
        pltpu.VMEM_SHARED((_N,), _f32),
        pltpu.SemaphoreType.DMA,
        pltpu.SemaphoreType.DMA,
        pltpu.SemaphoreType.DMA,
        pltpu.SemaphoreType.DMA,
        pltpu.SemaphoreType.DMA,
        pltpu.SemaphoreType.DMA,
    ],
    compiler_params=_sc_params,
)
def _seg(ei_hbm, ea_hbm, h1_hbm, h2_hbm, dg_hbm,
         dstv, srcv, ea0, ea1, onesv, zv1, h1s, h2s, dgs,
         isem0, isem1, ssem0, ssem1, dsem0, dsem1):
    cid = lax.axis_index("c")
    sid = lax.axis_index("s")
    eab = (ea0, ea1)
    isem = (isem0, isem1)
    ssem = (ssem0, ssem1)
    dsem = (dsem0, dsem1)

    @pl.when(cid == 0)
    def _():
        @pl.loop(0, _NT)
        def _(i):
            ea0[i, :] = jnp.zeros((_DE,), _f32)

        @pl.loop(0, 1024, step=16)
        def _(i):
            zv1[pl.ds(i, 16)] = jnp.zeros((16,), _f32)

        @pl.loop(0, _CH, step=16)
        def _(i):
            onesv[pl.ds(i, 16)] = jnp.ones((16,), _f32)

        nbase = sid * _NT
        pltpu.sync_copy(ea0.at[pl.ds(0, _NT)], h1s.at[pl.ds(nbase, _NT)])
        pltpu.sync_copy(ea0.at[pl.ds(0, _NT)], h2s.at[pl.ds(nbase, _NT)])

        @pl.when(sid < 10)
        def _():
            pltpu.sync_copy(zv1.at[pl.ds(0, 1000)], dgs.at[pl.ds(sid * 1000, 1000)])

        rbase = sid * _RT_A
        pltpu.sync_copy(ei_hbm.at[1, pl.ds(rbase, _RT_A)], dstv)
        pltpu.sync_copy(ei_hbm.at[0, pl.ds(rbase, _RT_A)], srcv)
        plsc.subcore_barrier()

        def fire_input(g, b):
            pltpu.async_copy(
                ea_hbm.at[pl.ds((rbase + g * _G1) * _CH, _GE1)], eab[b], isem[b])

        def drain_buf(b):
            pltpu.make_async_copy(
                ea_hbm.at[pl.ds(0, _GE1)], eab[b], isem[b]).wait()

        def fire_scat1(g, b):
            @pl.loop(0, _G1)
            def _(j):
                pltpu.async_copy(
                    eab[b].at[pl.ds(j * _CH, _CH)],
                    h1s.at[dstv.at[g * _G1 + j]], ssem[b], add=True)
                pltpu.async_copy(
                    onesv, dgs.at[dstv.at[g * _G1 + j]], dsem[b], add=True)

        def drain_scat1(b):
            @pl.loop(0, _G1)
            def _(j):
                pltpu.make_async_copy(
                    eab[b].at[pl.ds(0, _CH)], h1s.at[dstv.at[0]], ssem[b]).wait()
                pltpu.make_async_copy(
                    onesv, dgs.at[dstv.at[0]], dsem[b]).wait()

        fire_input(0, 0)
        for g in range(_NG1):
            b = g & 1
            if g + 1 < _NG1:
                if g >= 1:
                    drain_scat1(1 - b)
                fire_input(g + 1, 1 - b)
            drain_buf(b)
            fire_scat1(g, b)
        drain_scat1(0)
        drain_scat1(1)
        plsc.subcore_barrier()

        def fire_gat(g, b):
            @pl.loop(0, _G1)
            def _(j):
                pltpu.async_copy(
                    h1s.at[srcv.at[g * _G1 + j]],
                    eab[b].at[pl.ds(j * _CH, _CH)], isem[b])

        def fire_scat2(g, b):
            @pl.loop(0, _G1)
            def _(j):
                pltpu.async_copy(
                    eab[b].at[pl.ds(j * _CH, _CH)],
                    h2s.at[dstv.at[g * _G1 + j]], ssem[b], add=True)

        def drain_scat2(b):
            @pl.loop(0, _G1)
            def _(j):
                pltpu.make_async_copy(
                    eab[b].at[pl.ds(0, _CH)], h2s.at[dstv.at[0]], ssem[b]).wait()

        def drain_gat(b):
            @pl.loop(0, _G1)
            def _(j):
                pltpu.make_async_copy(
                    h1s.at[srcv.at[0]], eab[b].at[pl.ds(0, _CH)], isem[b]).wait()

        fire_gat(0, 0)
        for g in range(_NG1):
            b = g & 1
            if g + 1 < _NG1:
                if g >= 1:
                    drain_scat2(1 - b)
                fire_gat(g + 1, 1 - b)
            drain_gat(b)
            fire_scat2(g, b)
        drain_scat2(0)
        drain_scat2(1)
        plsc.subcore_barrier()

        st = ea0.at[pl.ds(0, _NT)]
        pltpu.sync_copy(h1s.at[pl.ds(nbase, _NT)], st)
        pltpu.sync_copy(st, h1_hbm.at[pl.ds(nbase, _NT)])
        pltpu.sync_copy(h2s.at[pl.ds(nbase, _NT)], st)
        pltpu.sync_copy(st, h2_hbm.at[pl.ds(nbase, _NT)])

        @pl.when(sid < 10)
        def _():
            pltpu.sync_copy(dgs.at[pl.ds(sid * 1000, 1000)], zv1.at[pl.ds(0, 1000)])
            pltpu.sync_copy(zv1.at[pl.ds(0, 1000)], dg_hbm.at[pl.ds(sid * 1000, 1000)])


def _mm_body(x_ref, h1_ref, h2_ref, dg_ref, wx_ref, w1_ref, w2_ref, b_ref, o_ref):
    norm = lax.rsqrt(jnp.maximum(dg_ref[:, 0:1], 1.0))
    acc = jnp.dot(x_ref[...], wx_ref[...],
                  preferred_element_type=_f32, precision=lax.Precision.HIGHEST)
    acc = acc + jnp.dot(h1_ref[...] * norm, w1_ref[...],
                        preferred_element_type=_f32, precision=lax.Precision.HIGHEST)
    acc = acc + jnp.dot(h2_ref[...] * norm, w2_ref[...],
                        preferred_element_type=_f32, precision=lax.Precision.HIGHEST)
    o_ref[...] = acc + b_ref[...]


_BLK = 2000


def _mm(x, h1, h2, dg, wx, w1, w2, b2):
    return pl.pallas_call(
        _mm_body,
        grid=(_N // _BLK,),
        in_specs=[
            pl.BlockSpec((_BLK, _DN), lambda i: (i, 0)),
            pl.BlockSpec((_BLK, _DE), lambda i: (i, 0)),
            pl.BlockSpec((_BLK, _DE), lambda i: (i, 0)),
            pl.BlockSpec((_BLK, 1), lambda i: (i, 0)),
            pl.BlockSpec((_DN, _DO), lambda i: (0, 0)),
            pl.BlockSpec((_DE, _DO), lambda i: (0, 0)),
            pl.BlockSpec((_DE, _DO), lambda i: (0, 0)),
            pl.BlockSpec((1, _DO), lambda i: (0, 0)),
        ],
        out_specs=pl.BlockSpec((_BLK, _DO), lambda i: (i, 0)),
        out_shape=jax.ShapeDtypeStruct((_N, _DO), _f32),
    )(x, h1, h2, dg, wx, w1, w2, b2)


_G = 5
_NG = _RT_C // _G
_GE = _G * _CH


@functools.partial(
    pl.kernel,
    mesh=_mesh,
    out_type=jax.ShapeDtypeStruct((_E, _DO), _f32),
    scratch_types=[
        pltpu.VMEM((_RT_C, _CH), jnp.int32),
        pltpu.VMEM((_GE, _DO), _f32),
        pltpu.VMEM((_GE, _DO), _f32),
        pltpu.SemaphoreType.DMA,
        pltpu.SemaphoreType.DMA,
        pltpu.SemaphoreType.DMA,
    ],
    compiler_params=_sc_params,
)
def _gat(ei_hbm, rst_hbm, est_hbm, srcv, buf0, buf1, gsem, wsem0, wsem1):
    wid = lax.axis_index("s") * _NC + lax.axis_index("c")
    rbase = wid * _RT_C
    pltpu.sync_copy(ei_hbm.at[0, pl.ds(rbase, _RT_C)], srcv)
    bufs = (buf0, buf1)
    wsems = (wsem0, wsem1)
    whandles = [None, None]
    for g in range(_NG):
        b = g % 2
        buf = bufs[b]
        if whandles[b] is not None:
            whandles[b].wait()
        ghandles = []
        for j in range(_G):
            ghandles.append(pltpu.async_copy(
                rst_hbm.at[srcv.at[g * _G + j]],
                buf.at[pl.ds(j * _CH, _CH)], gsem))
        for h in ghandles:
            h.wait()
        whandles[b] = pltpu.async_copy(
            buf, est_hbm.at[pl.ds((rbase + g * _G) * _CH, _GE)], wsems[b])
    whandles[0].wait()
    whandles[1].wait()


def kernel(x, edge_attr, W, b, edge_index):
    ei3 = edge_index.reshape(2, _ER, _CH)
    h1, h2, dg = _seg(ei3, edge_attr)
    wx = W[:, :_DN].T
    w1 = W[:, _DN:_DN + _DE].T
    w2 = W[:, _DN + _DE:].T
    rst = _mm(x, h1, h2, dg.reshape(_N, 1), wx, w1, w2, b.reshape(1, _DO))
    est = _gat(ei3, rst)
    return (rst, est)
